# baseline (speedup 1.0000x reference)
import functools

import jax
import jax.numpy as jnp
from jax import lax
from jax.experimental import pallas as pl
from jax.experimental.pallas import tpu as pltpu
from jax.experimental.pallas import tpu_sc as plsc

NUM_CORES = 2
NUM_SUBCORES = 16
K = 40
NBUF = 8
PREF = 5
LOOK = PREF - 1
BN_EPS = 1e-5


def _make_sc_aggregate(n, d, e):
    nw = NUM_CORES * NUM_SUBCORES
    assert e % (nw * K) == 0 and n % NUM_SUBCORES == 0
    epw = e // nw
    n_chunks = epw // K
    rpt = n // NUM_SUBCORES
    assert n_chunks > 2 * NBUF and 1 <= LOOK < PREF < NBUF

    mesh = plsc.VectorSubcoreMesh(
        core_axis_name="c", subcore_axis_name="s",
        num_cores=NUM_CORES, num_subcores=NUM_SUBCORES)

    @functools.partial(
        pl.kernel,
        out_type=jax.ShapeDtypeStruct((NUM_CORES, n, d), jnp.float32),
        mesh=mesh,
        scratch_types=[
            pltpu.VMEM_SHARED((n, d), jnp.float32),
            pltpu.VMEM((NBUF, K), jnp.int32),
            pltpu.VMEM((NBUF, K), jnp.int32),
            pltpu.VMEM((NBUF, K, d), jnp.float32),
            pltpu.SemaphoreType.DMA((NBUF,)),
            pltpu.SemaphoreType.DMA((NBUF,)),
            pltpu.SemaphoreType.DMA((NBUF,)),
        ],
        compiler_params=pltpu.CompilerParams(use_tc_tiling_on_sc=False),
    )
    def agg(h_hbm, ei_hbm, part_hbm,
            accum, srci, dsti, rows, gsem, dsem, ssem):
        c = lax.axis_index("c")
        s = lax.axis_index("s")
        row0 = s * rpt
        e_base = (c * NUM_SUBCORES + s) * epw

        @pl.when(c == 0)
        def _():
            pltpu.sync_copy(h_hbm.at[pl.ds(row0, rpt)], accum.at[pl.ds(row0, rpt)])

        @pl.when(c != 0)
        def _():
            zv = jnp.zeros((16,), jnp.float32)

            def zero_vec(i, _):
                rows[0, i // (d // 16), pl.ds((i % (d // 16)) * 16, 16)] = zv
                return 0
            lax.fori_loop(0, K * d // 16, zero_vec, 0)
            nfull = rpt // K
            for q in range(nfull):
                pltpu.sync_copy(rows.at[0], accum.at[pl.ds(row0 + q * K, K)])
            rem = rpt - nfull * K
            if rem:
                pltpu.sync_copy(rows.at[0].at[pl.ds(0, rem)],
                                accum.at[pl.ds(row0 + nfull * K, rem)])

        plsc.subcore_barrier()

        def issue_pref(j, b):
            pltpu.async_copy(ei_hbm.at[0, pl.ds(e_base + j * K, K)],
                             srci.at[b], dsem.at[b])
            pltpu.async_copy(ei_hbm.at[1, pl.ds(e_base + j * K, K)],
                             dsti.at[b], dsem.at[b])

        def issue_gather(j, b):
            pltpu.async_copy(h_hbm.at[srci.at[b]], rows.at[b], gsem.at[b])

        def wait_pref(b):
            pltpu.make_async_copy(ei_hbm.at[0, pl.ds(0, K)], srci.at[b],
                                  dsem.at[b]).wait()
            pltpu.make_async_copy(ei_hbm.at[1, pl.ds(0, K)], dsti.at[b],
                                  dsem.at[b]).wait()

        def wait_gather(b):
            pltpu.make_async_copy(h_hbm.at[pl.ds(0, K)], rows.at[b],
                                  gsem.at[b]).wait()

        def issue_scatter(b):
            pltpu.async_copy(rows.at[b], accum.at[dsti.at[b]], ssem.at[b],
                             add=True)

        def wait_scatter(b):
            pltpu.make_async_copy(rows.at[b], accum.at[pl.ds(0, K)],
                                  ssem.at[b]).wait()

        def iteration(j, b, bp, bg, wait_s, do_pref, do_gather):
            wait_gather(b)
            issue_scatter(b)
            if wait_s:
                wait_scatter(bp)
            if do_pref:
                issue_pref(j + PREF, bp)
            if do_gather:
                wait_pref(bg)
                issue_gather(j + LOOK, bg)

        for j in range(PREF):
            issue_pref(j, j)
        for j in range(LOOK):
            wait_pref(j)
            issue_gather(j, j)
        for j in range(NBUF - PREF):
            iteration(j, j % NBUF, (j + PREF) % NBUF, (j + LOOK) % NBUF,
                      False, True, True)

        def body(j, _):
            iteration(j, lax.rem(j, NBUF), lax.rem(j + PREF, NBUF),
                      lax.rem(j + LOOK, NBUF), True, True, True)
            return 0
        lax.fori_loop(NBUF - PREF, n_chunks - PREF, body, 0)

        for j in range(n_chunks - PREF, n_chunks - LOOK):
            iteration(j, j % NBUF, (j + PREF) % NBUF, (j + LOOK) % NBUF,
                      False, False, True)
        for j in range(n_chunks - LOOK, n_chunks):
            iteration(j, j % NBUF, 0, 0, False, False, False)
        for j in range(n_chunks - NBUF, n_chunks):
            wait_scatter(j % NBUF)

        plsc.subcore_barrier()

        pltpu.sync_copy(accum.at[pl.ds(row0, rpt)], part_hbm.at[c, pl.ds(row0, rpt)])

    return agg


def _mlp_block(p_ref, w1_ref, g1_ref, b1_ref, e1_ref,
               w2_ref, g2_ref, b2_ref, e2_ref, o_ref):
    inv = 1.0 / jnp.sqrt(1.0 + BN_EPS)
    z = p_ref[0] + p_ref[1]
    s1 = g1_ref[...] * inv
    h1 = jnp.dot(z, w1_ref[...], preferred_element_type=jnp.float32) * s1
    h1 = jnp.maximum(h1 + (b1_ref[...] * s1 + e1_ref[...]), 0.0)
    s2 = g2_ref[...] * inv
    h2 = jnp.dot(h1, w2_ref[...], preferred_element_type=jnp.float32) * s2
    o_ref[...] = jnp.maximum(h2 + (b2_ref[...] * s2 + e2_ref[...]), 0.0)


def _make_tc_mlp(n, d, block_rows):
    assert n % block_rows == 0
    grid = (n // block_rows,)
    mat = pl.BlockSpec((d, d), lambda i: (0, 0))
    vec = pl.BlockSpec((1, d), lambda i: (0, 0))
    return pl.pallas_call(
        _mlp_block,
        grid=grid,
        in_specs=[pl.BlockSpec((2, block_rows, d), lambda i: (0, i, 0)),
                  mat, vec, vec, vec, mat, vec, vec, vec],
        out_specs=pl.BlockSpec((block_rows, d), lambda i: (i, 0)),
        out_shape=jax.ShapeDtypeStruct((n, d), jnp.float32),
    )


def kernel(x, edge_index, params):
    n, d = x.shape
    e = edge_index.shape[1]
    ei = edge_index.astype(jnp.int32)

    agg = _make_sc_aggregate(n, d, e)
    mlp = _make_tc_mlp(n, d, 2000)

    h = x
    for i in range(3):
        part = agg(h, ei)
        h = mlp(part,
                params[f"W1_{i}"], params[f"g1_{i}"].reshape(1, d),
                params[f"b1_{i}"].reshape(1, d), params[f"be1_{i}"].reshape(1, d),
                params[f"W2_{i}"], params[f"g2_{i}"].reshape(1, d),
                params[f"b2_{i}"].reshape(1, d), params[f"be2_{i}"].reshape(1, d))
    return h


# speedup vs baseline: 1.0145x; 1.0145x over previous
import functools

import jax
import jax.numpy as jnp
from jax import lax
from jax.experimental import pallas as pl
from jax.experimental.pallas import tpu as pltpu
from jax.experimental.pallas import tpu_sc as plsc

NUM_CORES = 2
NUM_SUBCORES = 16
K = 80
NBUF = 4
PREF = 3
LOOK = PREF - 1
BN_EPS = 1e-5


def _make_sc_aggregate(n, d, e):
    nw = NUM_CORES * NUM_SUBCORES
    assert e % (nw * K) == 0 and n % NUM_SUBCORES == 0
    epw = e // nw
    n_chunks = epw // K
    rpt = n // NUM_SUBCORES
    assert n_chunks > 2 * NBUF and 1 <= LOOK < PREF < NBUF

    mesh = plsc.VectorSubcoreMesh(
        core_axis_name="c", subcore_axis_name="s",
        num_cores=NUM_CORES, num_subcores=NUM_SUBCORES)

    @functools.partial(
        pl.kernel,
        out_type=jax.ShapeDtypeStruct((NUM_CORES, n, d), jnp.float32),
        mesh=mesh,
        scratch_types=[
            pltpu.VMEM_SHARED((n, d), jnp.float32),
            pltpu.VMEM((NBUF, K), jnp.int32),
            pltpu.VMEM((NBUF, K), jnp.int32),
            pltpu.VMEM((NBUF, K, d), jnp.float32),
            pltpu.SemaphoreType.DMA((NBUF,)),
            pltpu.SemaphoreType.DMA((NBUF,)),
            pltpu.SemaphoreType.DMA((NBUF,)),
        ],
        compiler_params=pltpu.CompilerParams(use_tc_tiling_on_sc=False),
    )
    def agg(h_hbm, ei_hbm, part_hbm,
            accum, srci, dsti, rows, gsem, dsem, ssem):
        c = lax.axis_index("c")
        s = lax.axis_index("s")
        row0 = s * rpt
        e_base = (c * NUM_SUBCORES + s) * epw

        @pl.when(c == 0)
        def _():
            pltpu.sync_copy(h_hbm.at[pl.ds(row0, rpt)], accum.at[pl.ds(row0, rpt)])

        @pl.when(c != 0)
        def _():
            zv = jnp.zeros((16,), jnp.float32)

            def zero_vec(i, _):
                rows[0, i // (d // 16), pl.ds((i % (d // 16)) * 16, 16)] = zv
                return 0
            lax.fori_loop(0, K * d // 16, zero_vec, 0)
            nfull = rpt // K
            for q in range(nfull):
                pltpu.sync_copy(rows.at[0], accum.at[pl.ds(row0 + q * K, K)])
            rem = rpt - nfull * K
            if rem:
                pltpu.sync_copy(rows.at[0].at[pl.ds(0, rem)],
                                accum.at[pl.ds(row0 + nfull * K, rem)])

        plsc.subcore_barrier()

        def issue_pref(j, b):
            pltpu.async_copy(ei_hbm.at[0, pl.ds(e_base + j * K, K)],
                             srci.at[b], dsem.at[b])
            pltpu.async_copy(ei_hbm.at[1, pl.ds(e_base + j * K, K)],
                             dsti.at[b], dsem.at[b])

        def issue_gather(j, b):
            pltpu.async_copy(h_hbm.at[srci.at[b]], rows.at[b], gsem.at[b])

        def wait_pref(b):
            pltpu.make_async_copy(ei_hbm.at[0, pl.ds(0, K)], srci.at[b],
                                  dsem.at[b]).wait()
            pltpu.make_async_copy(ei_hbm.at[1, pl.ds(0, K)], dsti.at[b],
                                  dsem.at[b]).wait()

        def wait_gather(b):
            pltpu.make_async_copy(h_hbm.at[pl.ds(0, K)], rows.at[b],
                                  gsem.at[b]).wait()

        def issue_scatter(b):
            pltpu.async_copy(rows.at[b], accum.at[dsti.at[b]], ssem.at[b],
                             add=True)

        def wait_scatter(b):
            pltpu.make_async_copy(rows.at[b], accum.at[pl.ds(0, K)],
                                  ssem.at[b]).wait()

        def iteration(j, b, bp, bg, wait_s, do_pref, do_gather):
            wait_gather(b)
            issue_scatter(b)
            if wait_s:
                wait_scatter(bp)
            if do_pref:
                issue_pref(j + PREF, bp)
            if do_gather:
                wait_pref(bg)
                issue_gather(j + LOOK, bg)

        for j in range(PREF):
            issue_pref(j, j)
        for j in range(LOOK):
            wait_pref(j)
            issue_gather(j, j)
        for j in range(NBUF - PREF):
            iteration(j, j % NBUF, (j + PREF) % NBUF, (j + LOOK) % NBUF,
                      False, True, True)

        def body(j, _):
            iteration(j, lax.rem(j, NBUF), lax.rem(j + PREF, NBUF),
                      lax.rem(j + LOOK, NBUF), True, True, True)
            return 0
        lax.fori_loop(NBUF - PREF, n_chunks - PREF, body, 0)

        for j in range(n_chunks - PREF, n_chunks - LOOK):
            iteration(j, j % NBUF, (j + PREF) % NBUF, (j + LOOK) % NBUF,
                      False, False, True)
        for j in range(n_chunks - LOOK, n_chunks):
            iteration(j, j % NBUF, 0, 0, False, False, False)
        for j in range(n_chunks - NBUF, n_chunks):
            wait_scatter(j % NBUF)

        plsc.subcore_barrier()

        pltpu.sync_copy(accum.at[pl.ds(row0, rpt)], part_hbm.at[c, pl.ds(row0, rpt)])

    return agg


def _mlp_block(p_ref, w1_ref, g1_ref, b1_ref, e1_ref,
               w2_ref, g2_ref, b2_ref, e2_ref, o_ref):
    inv = 1.0 / jnp.sqrt(1.0 + BN_EPS)
    z = p_ref[0] + p_ref[1]
    s1 = g1_ref[...] * inv
    h1 = jnp.dot(z, w1_ref[...], preferred_element_type=jnp.float32) * s1
    h1 = jnp.maximum(h1 + (b1_ref[...] * s1 + e1_ref[...]), 0.0)
    s2 = g2_ref[...] * inv
    h2 = jnp.dot(h1, w2_ref[...], preferred_element_type=jnp.float32) * s2
    o_ref[...] = jnp.maximum(h2 + (b2_ref[...] * s2 + e2_ref[...]), 0.0)


def _make_tc_mlp(n, d, block_rows):
    assert n % block_rows == 0
    grid = (n // block_rows,)
    mat = pl.BlockSpec((d, d), lambda i: (0, 0))
    vec = pl.BlockSpec((1, d), lambda i: (0, 0))
    return pl.pallas_call(
        _mlp_block,
        grid=grid,
        in_specs=[pl.BlockSpec((2, block_rows, d), lambda i: (0, i, 0)),
                  mat, vec, vec, vec, mat, vec, vec, vec],
        out_specs=pl.BlockSpec((block_rows, d), lambda i: (i, 0)),
        out_shape=jax.ShapeDtypeStruct((n, d), jnp.float32),
    )


def kernel(x, edge_index, params):
    n, d = x.shape
    e = edge_index.shape[1]
    ei = edge_index.astype(jnp.int32)

    agg = _make_sc_aggregate(n, d, e)
    mlp = _make_tc_mlp(n, d, 2000)

    h = x
    for i in range(3):
        part = agg(h, ei)
        h = mlp(part,
                params[f"W1_{i}"], params[f"g1_{i}"].reshape(1, d),
                params[f"b1_{i}"].reshape(1, d), params[f"be1_{i}"].reshape(1, d),
                params[f"W2_{i}"], params[f"g2_{i}"].reshape(1, d),
                params[f"b2_{i}"].reshape(1, d), params[f"be2_{i}"].reshape(1, d))
    return h
